# baseline (speedup 1.0000x reference)
import functools

import jax
import jax.numpy as jnp
from jax import lax
from jax.experimental import pallas as pl
from jax.experimental.pallas import tpu as pltpu
from jax.experimental.pallas import tpu_sc as plsc

_N, _E, _F_IN, _H = 10000, 320000, 128, 64

_NC, _NS = 2, 16
_NW = _NC * _NS
_CH = 128
_EPW = 10240
_NCHUNK = _EPW // _CH
_EPAD = _NW * _EPW
_NPAD = 10240
_NG = 8
_RPT = _NPAD // _NS
_BLK = 640

_mesh = plsc.VectorSubcoreMesh(core_axis_name="c", subcore_axis_name="s")


@functools.partial(
    pl.kernel,
    out_type=jax.ShapeDtypeStruct((_NW, _NPAD), jnp.float32),
    mesh=_mesh,
    compiler_params=pltpu.CompilerParams(needs_layout_passes=False),
    scratch_types=[
        pltpu.VMEM((_EPW,), jnp.int32),
        pltpu.VMEM((_NPAD,), jnp.float32),
    ],
)
def _deg_kernel(dst_hbm, zeros_hbm, out_hbm, dst_v, deg_v):
    c = lax.axis_index("c")
    s = lax.axis_index("s")
    w = s * _NC + c
    pltpu.sync_copy(dst_hbm.at[w], dst_v)
    pltpu.sync_copy(zeros_hbm, deg_v)
    ones = jnp.ones((16,), jnp.float32)

    def body(i, carry):
        idx = dst_v[pl.ds(i * 16, 16)]
        plsc.addupdate_scatter(deg_v, [idx], ones)
        return carry

    lax.fori_loop(0, _EPW // 16, body, 0)
    pltpu.sync_copy(deg_v, out_hbm.at[w])


@functools.partial(
    pl.kernel,
    out_type=jax.ShapeDtypeStruct((_NC, _NPAD, _H), jnp.float32),
    mesh=_mesh,
    compiler_params=pltpu.CompilerParams(
        needs_layout_passes=False, use_tc_tiling_on_sc=False),
    scratch_types=[
        pltpu.VMEM((_NCHUNK, _CH), jnp.int32),
        pltpu.VMEM((_NCHUNK, _CH), jnp.int32),
        pltpu.VMEM((_NG, _CH, _H // 2), jnp.int32),
        pltpu.VMEM((2, _CH, _H), jnp.float32),
        pltpu.VMEM_SHARED((_NPAD, _H), jnp.float32),
    ] + [pltpu.SemaphoreType.DMA] * (_NG + 2),
)
def _spmm_kernel(src_hbm, dst_hbm, hsb_hbm, zrows_hbm, out_hbm,
                 src_v, dst_v, rows_b, rows_f, acc_sh, *sems):
    c = lax.axis_index("c")
    s = lax.axis_index("s")
    w = s * _NC + c
    pltpu.sync_copy(src_hbm.at[w], src_v)
    pltpu.sync_copy(dst_hbm.at[w], dst_v)
    pltpu.sync_copy(zrows_hbm, acc_sh.at[pl.ds(s * _RPT, _RPT)])
    plsc.subcore_barrier()

    gsems = sems[:_NG]
    ssems = sems[_NG:]

    def fire_g(buf, g):
        pltpu.async_copy(hsb_hbm.at[src_v.at[g]], rows_b.at[buf], gsems[buf])

    def drain_g(buf, g):
        pltpu.make_async_copy(hsb_hbm.at[src_v.at[g]], rows_b.at[buf],
                              gsems[buf]).wait()

    def unpack(buf, sbuf):
        rb = rows_b.at[buf]
        rf = rows_f.at[sbuf]

        @pl.loop(0, _CH, unroll=8)
        def _rows(r):
            for g in range(_H // 32):
                x = rb[r, pl.ds(g * 16, 16)]
                a = plsc.bitcast(x << 16, jnp.float32)
                b = plsc.bitcast(x & jnp.int32(-65536), jnp.float32)
                rf[r, pl.ds(g * 32, 16)] = a
                rf[r, pl.ds(g * 32 + 16, 16)] = b

    def fire_s(sbuf, g):
        pltpu.async_copy(rows_f.at[sbuf], acc_sh.at[dst_v.at[g]], ssems[sbuf],
                         add=True)

    def drain_s(sbuf, g):
        pltpu.make_async_copy(rows_f.at[sbuf], acc_sh.at[dst_v.at[g]],
                              ssems[sbuf]).wait()

    for i in range(_NG):
        fire_g(i, i)

    @pl.loop(0, _NCHUNK, step=_NG)
    def _pipeline(t):
        for i in range(_NG):
            g = t + i
            drain_g(i, g)

            @pl.when(g >= 2)
            def _():
                drain_s(i % 2, g - 2)

            unpack(i, i % 2)
            fire_s(i % 2, g)

            @pl.when(g + _NG < _NCHUNK)
            def _():
                fire_g(i, g + _NG)

    drain_s(0, _NCHUNK - 2)
    drain_s(1, _NCHUNK - 1)
    plsc.subcore_barrier()
    pltpu.sync_copy(acc_sh.at[pl.ds(s * _RPT, _RPT)],
                    out_hbm.at[c, pl.ds(s * _RPT, _RPT)])


def _shuffle_bf16(x):
    r = x.shape[0]
    y = x.reshape(r, 2, 2, 16)
    ua = lax.bitcast_convert_type(y[:, :, 0, :], jnp.int32)
    ub = lax.bitcast_convert_type(y[:, :, 1, :], jnp.int32)

    def rne16(u):
        return ((u + 0x7FFF + ((u >> 16) & 1)) >> 16) & 0xFFFF

    packed = rne16(ua) | (rne16(ub) << 16)
    return packed.reshape(r, _H // 2)


def _first_body(deg_ref, x_ref, w_ref, dinv_ref, hs_ref, hsb_ref):
    deg = jnp.sum(deg_ref[...], axis=0) + 1.0
    dinv = lax.rsqrt(deg)
    dinvb = jnp.broadcast_to(dinv[:, None], (_BLK, _H))
    dinv_ref[...] = dinvb
    h = jnp.dot(x_ref[...], w_ref[...], preferred_element_type=jnp.float32)
    hs = h * dinvb
    hs_ref[...] = hs
    hsb_ref[...] = _shuffle_bf16(hs)


def _mid_body(acc_ref, hs_ref, dinv_ref, w_ref, b_ref, out_ref, hsb_ref):
    acc = acc_ref[...]
    z = (acc[0] + acc[1] + hs_ref[...]) * dinv_ref[...] + b_ref[...]
    z = jnp.maximum(z, 0.0)
    hsn = (jnp.dot(z, w_ref[...], preferred_element_type=jnp.float32)
           * dinv_ref[...])
    out_ref[...] = hsn
    hsb_ref[...] = _shuffle_bf16(hsn)


def _fin_body(acc_ref, hs_ref, dinv_ref, b_ref, out_ref):
    acc = acc_ref[...]
    z = (acc[0] + acc[1] + hs_ref[...]) * dinv_ref[...] + b_ref[...]
    m = jnp.max(z, axis=1, keepdims=True)
    e = jnp.exp(z - m)
    lse = jnp.log(jnp.sum(e, axis=1, keepdims=True))
    out_ref[...] = z - m - lse


def _tc_first(deg_parts, feat, w1):
    return pl.pallas_call(
        _first_body,
        grid=(_NPAD // _BLK,),
        in_specs=[
            pl.BlockSpec((_NW, _BLK), lambda i: (0, i)),
            pl.BlockSpec((_BLK, _F_IN), lambda i: (i, 0)),
            pl.BlockSpec((_F_IN, _H), lambda i: (0, 0)),
        ],
        out_specs=[
            pl.BlockSpec((_BLK, _H), lambda i: (i, 0)),
            pl.BlockSpec((_BLK, _H), lambda i: (i, 0)),
            pl.BlockSpec((_BLK, _H // 2), lambda i: (i, 0)),
        ],
        out_shape=[
            jax.ShapeDtypeStruct((_NPAD, _H), jnp.float32),
            jax.ShapeDtypeStruct((_NPAD, _H), jnp.float32),
            jax.ShapeDtypeStruct((_NPAD, _H // 2), jnp.int32),
        ],
    )(deg_parts, feat, w1)


def _tc_mid(acc, hs, dinvb, w, b):
    return pl.pallas_call(
        _mid_body,
        grid=(_NPAD // _BLK,),
        in_specs=[
            pl.BlockSpec((_NC, _BLK, _H), lambda i: (0, i, 0)),
            pl.BlockSpec((_BLK, _H), lambda i: (i, 0)),
            pl.BlockSpec((_BLK, _H), lambda i: (i, 0)),
            pl.BlockSpec((_H, _H), lambda i: (0, 0)),
            pl.BlockSpec((1, _H), lambda i: (0, 0)),
        ],
        out_specs=[
            pl.BlockSpec((_BLK, _H), lambda i: (i, 0)),
            pl.BlockSpec((_BLK, _H // 2), lambda i: (i, 0)),
        ],
        out_shape=[
            jax.ShapeDtypeStruct((_NPAD, _H), jnp.float32),
            jax.ShapeDtypeStruct((_NPAD, _H // 2), jnp.int32),
        ],
    )(acc, hs, dinvb, w, b)


def _tc_fin(acc, hs, dinvb, b):
    return pl.pallas_call(
        _fin_body,
        grid=(_NPAD // _BLK,),
        in_specs=[
            pl.BlockSpec((_NC, _BLK, _H), lambda i: (0, i, 0)),
            pl.BlockSpec((_BLK, _H), lambda i: (i, 0)),
            pl.BlockSpec((_BLK, _H), lambda i: (i, 0)),
            pl.BlockSpec((1, _H), lambda i: (0, 0)),
        ],
        out_specs=pl.BlockSpec((_BLK, _H), lambda i: (i, 0)),
        out_shape=jax.ShapeDtypeStruct((_NPAD, _H), jnp.float32),
    )(acc, hs, dinvb, b)


def kernel(edges, features, W1, b1, W2, b2, W3, b3):
    src = edges[0].astype(jnp.int32)
    dst = edges[1].astype(jnp.int32)
    src_p = jnp.concatenate(
        [src, jnp.zeros((_EPAD - _E,), jnp.int32)]).reshape(_NW, _NCHUNK, _CH)
    dst_p = jnp.concatenate(
        [dst, jnp.full((_EPAD - _E,), _N, jnp.int32)]).reshape(_NW, _NCHUNK, _CH)
    dst_flat = dst_p.reshape(_NW, _EPW)
    zeros_n = jnp.zeros((_NPAD,), jnp.float32)
    zrows = jnp.zeros((_RPT, _H), jnp.float32)
    feat_pad = jnp.concatenate(
        [features, jnp.zeros((_NPAD - _N, _F_IN), jnp.float32)], axis=0)

    deg_parts = _deg_kernel(dst_flat, zeros_n)
    dinvb, hs, hsb = _tc_first(deg_parts, feat_pad, W1)

    acc = _spmm_kernel(src_p, dst_p, hsb, zrows)
    hs, hsb = _tc_mid(acc, hs, dinvb, W2, b1.reshape(1, _H))

    acc = _spmm_kernel(src_p, dst_p, hsb, zrows)
    hs, hsb = _tc_mid(acc, hs, dinvb, W3, b2.reshape(1, _H))

    acc = _spmm_kernel(src_p, dst_p, hsb, zrows)
    out = _tc_fin(acc, hs, dinvb, b3.reshape(1, _H))
    return out[:_N]


# speedup vs baseline: 1.0062x; 1.0062x over previous
import functools

import jax
import jax.numpy as jnp
from jax import lax
from jax.experimental import pallas as pl
from jax.experimental.pallas import tpu as pltpu
from jax.experimental.pallas import tpu_sc as plsc

_N, _E, _F_IN, _H = 10000, 320000, 128, 64

_NC, _NS = 2, 16
_NW = _NC * _NS
_CH = 128
_EPW = 10240
_NCHUNK = _EPW // _CH
_EPAD = _NW * _EPW
_NPAD = 10240
_NG = 10
_RPT = _NPAD // _NS
_BLK = 640

_mesh = plsc.VectorSubcoreMesh(core_axis_name="c", subcore_axis_name="s")


@functools.partial(
    pl.kernel,
    out_type=jax.ShapeDtypeStruct((_NW, _NPAD), jnp.float32),
    mesh=_mesh,
    compiler_params=pltpu.CompilerParams(needs_layout_passes=False),
    scratch_types=[
        pltpu.VMEM((_EPW,), jnp.int32),
        pltpu.VMEM((_NPAD,), jnp.float32),
    ],
)
def _deg_kernel(dst_hbm, zeros_hbm, out_hbm, dst_v, deg_v):
    c = lax.axis_index("c")
    s = lax.axis_index("s")
    w = s * _NC + c
    pltpu.sync_copy(dst_hbm.at[w], dst_v)
    pltpu.sync_copy(zeros_hbm, deg_v)
    ones = jnp.ones((16,), jnp.float32)

    def body(i, carry):
        idx = dst_v[pl.ds(i * 16, 16)]
        plsc.addupdate_scatter(deg_v, [idx], ones)
        return carry

    lax.fori_loop(0, _EPW // 16, body, 0)
    pltpu.sync_copy(deg_v, out_hbm.at[w])


@functools.partial(
    pl.kernel,
    out_type=jax.ShapeDtypeStruct((_NC, _NPAD, _H), jnp.float32),
    mesh=_mesh,
    compiler_params=pltpu.CompilerParams(
        needs_layout_passes=False, use_tc_tiling_on_sc=False),
    scratch_types=[
        pltpu.VMEM((_NCHUNK, _CH), jnp.int32),
        pltpu.VMEM((_NCHUNK, _CH), jnp.int32),
        pltpu.VMEM((_NG, _CH, _H // 2), jnp.int32),
        pltpu.VMEM((2, _CH, _H), jnp.float32),
        pltpu.VMEM_SHARED((_NPAD, _H), jnp.float32),
    ] + [pltpu.SemaphoreType.DMA] * (_NG + 2),
)
def _spmm_kernel(src_hbm, dst_hbm, hsb_hbm, zrows_hbm, out_hbm,
                 src_v, dst_v, rows_b, rows_f, acc_sh, *sems):
    c = lax.axis_index("c")
    s = lax.axis_index("s")
    w = s * _NC + c
    pltpu.sync_copy(src_hbm.at[w], src_v)
    pltpu.sync_copy(dst_hbm.at[w], dst_v)
    pltpu.sync_copy(zrows_hbm, acc_sh.at[pl.ds(s * _RPT, _RPT)])
    plsc.subcore_barrier()

    gsems = sems[:_NG]
    ssems = sems[_NG:]

    def fire_g(buf, g):
        pltpu.async_copy(hsb_hbm.at[src_v.at[g]], rows_b.at[buf], gsems[buf])

    def drain_g(buf, g):
        pltpu.make_async_copy(hsb_hbm.at[src_v.at[g]], rows_b.at[buf],
                              gsems[buf]).wait()

    def unpack(buf, sbuf):
        rb = rows_b.at[buf]
        rf = rows_f.at[sbuf]

        @pl.loop(0, _CH, unroll=8)
        def _rows(r):
            for g in range(_H // 32):
                x = rb[r, pl.ds(g * 16, 16)]
                a = plsc.bitcast(x << 16, jnp.float32)
                b = plsc.bitcast(x & jnp.int32(-65536), jnp.float32)
                rf[r, pl.ds(g * 32, 16)] = a
                rf[r, pl.ds(g * 32 + 16, 16)] = b

    def fire_s(sbuf, g):
        pltpu.async_copy(rows_f.at[sbuf], acc_sh.at[dst_v.at[g]], ssems[sbuf],
                         add=True)

    def drain_s(sbuf, g):
        pltpu.make_async_copy(rows_f.at[sbuf], acc_sh.at[dst_v.at[g]],
                              ssems[sbuf]).wait()

    for i in range(_NG):
        fire_g(i, i)

    @pl.loop(0, _NCHUNK, step=_NG)
    def _pipeline(t):
        for i in range(_NG):
            g = t + i
            drain_g(i, g)

            @pl.when(g >= 2)
            def _():
                drain_s(i % 2, g - 2)

            unpack(i, i % 2)
            fire_s(i % 2, g)

            @pl.when(g + _NG < _NCHUNK)
            def _():
                fire_g(i, g + _NG)

    drain_s(0, _NCHUNK - 2)
    drain_s(1, _NCHUNK - 1)
    plsc.subcore_barrier()
    pltpu.sync_copy(acc_sh.at[pl.ds(s * _RPT, _RPT)],
                    out_hbm.at[c, pl.ds(s * _RPT, _RPT)])


def _shuffle_bf16(x):
    r = x.shape[0]
    y = x.reshape(r, 2, 2, 16)
    ua = lax.bitcast_convert_type(y[:, :, 0, :], jnp.int32)
    ub = lax.bitcast_convert_type(y[:, :, 1, :], jnp.int32)

    def rne16(u):
        return ((u + 0x7FFF + ((u >> 16) & 1)) >> 16) & 0xFFFF

    packed = rne16(ua) | (rne16(ub) << 16)
    return packed.reshape(r, _H // 2)


def _first_body(deg_ref, x_ref, w_ref, dinv_ref, hs_ref, hsb_ref):
    deg = jnp.sum(deg_ref[...], axis=0) + 1.0
    dinv = lax.rsqrt(deg)
    dinvb = jnp.broadcast_to(dinv[:, None], (_BLK, _H))
    dinv_ref[...] = dinvb
    h = jnp.dot(x_ref[...], w_ref[...], preferred_element_type=jnp.float32)
    hs = h * dinvb
    hs_ref[...] = hs
    hsb_ref[...] = _shuffle_bf16(hs)


def _mid_body(acc_ref, hs_ref, dinv_ref, w_ref, b_ref, out_ref, hsb_ref):
    acc = acc_ref[...]
    z = (acc[0] + acc[1] + hs_ref[...]) * dinv_ref[...] + b_ref[...]
    z = jnp.maximum(z, 0.0)
    hsn = (jnp.dot(z, w_ref[...], preferred_element_type=jnp.float32)
           * dinv_ref[...])
    out_ref[...] = hsn
    hsb_ref[...] = _shuffle_bf16(hsn)


def _fin_body(acc_ref, hs_ref, dinv_ref, b_ref, out_ref):
    acc = acc_ref[...]
    z = (acc[0] + acc[1] + hs_ref[...]) * dinv_ref[...] + b_ref[...]
    m = jnp.max(z, axis=1, keepdims=True)
    e = jnp.exp(z - m)
    lse = jnp.log(jnp.sum(e, axis=1, keepdims=True))
    out_ref[...] = z - m - lse


def _tc_first(deg_parts, feat, w1):
    return pl.pallas_call(
        _first_body,
        grid=(_NPAD // _BLK,),
        in_specs=[
            pl.BlockSpec((_NW, _BLK), lambda i: (0, i)),
            pl.BlockSpec((_BLK, _F_IN), lambda i: (i, 0)),
            pl.BlockSpec((_F_IN, _H), lambda i: (0, 0)),
        ],
        out_specs=[
            pl.BlockSpec((_BLK, _H), lambda i: (i, 0)),
            pl.BlockSpec((_BLK, _H), lambda i: (i, 0)),
            pl.BlockSpec((_BLK, _H // 2), lambda i: (i, 0)),
        ],
        out_shape=[
            jax.ShapeDtypeStruct((_NPAD, _H), jnp.float32),
            jax.ShapeDtypeStruct((_NPAD, _H), jnp.float32),
            jax.ShapeDtypeStruct((_NPAD, _H // 2), jnp.int32),
        ],
    )(deg_parts, feat, w1)


def _tc_mid(acc, hs, dinvb, w, b):
    return pl.pallas_call(
        _mid_body,
        grid=(_NPAD // _BLK,),
        in_specs=[
            pl.BlockSpec((_NC, _BLK, _H), lambda i: (0, i, 0)),
            pl.BlockSpec((_BLK, _H), lambda i: (i, 0)),
            pl.BlockSpec((_BLK, _H), lambda i: (i, 0)),
            pl.BlockSpec((_H, _H), lambda i: (0, 0)),
            pl.BlockSpec((1, _H), lambda i: (0, 0)),
        ],
        out_specs=[
            pl.BlockSpec((_BLK, _H), lambda i: (i, 0)),
            pl.BlockSpec((_BLK, _H // 2), lambda i: (i, 0)),
        ],
        out_shape=[
            jax.ShapeDtypeStruct((_NPAD, _H), jnp.float32),
            jax.ShapeDtypeStruct((_NPAD, _H // 2), jnp.int32),
        ],
    )(acc, hs, dinvb, w, b)


def _tc_fin(acc, hs, dinvb, b):
    return pl.pallas_call(
        _fin_body,
        grid=(_NPAD // _BLK,),
        in_specs=[
            pl.BlockSpec((_NC, _BLK, _H), lambda i: (0, i, 0)),
            pl.BlockSpec((_BLK, _H), lambda i: (i, 0)),
            pl.BlockSpec((_BLK, _H), lambda i: (i, 0)),
            pl.BlockSpec((1, _H), lambda i: (0, 0)),
        ],
        out_specs=pl.BlockSpec((_BLK, _H), lambda i: (i, 0)),
        out_shape=jax.ShapeDtypeStruct((_NPAD, _H), jnp.float32),
    )(acc, hs, dinvb, b)


def kernel(edges, features, W1, b1, W2, b2, W3, b3):
    src = edges[0].astype(jnp.int32)
    dst = edges[1].astype(jnp.int32)
    src_p = jnp.concatenate(
        [src, jnp.zeros((_EPAD - _E,), jnp.int32)]).reshape(_NW, _NCHUNK, _CH)
    dst_p = jnp.concatenate(
        [dst, jnp.full((_EPAD - _E,), _N, jnp.int32)]).reshape(_NW, _NCHUNK, _CH)
    dst_flat = dst_p.reshape(_NW, _EPW)
    zeros_n = jnp.zeros((_NPAD,), jnp.float32)
    zrows = jnp.zeros((_RPT, _H), jnp.float32)
    feat_pad = jnp.concatenate(
        [features, jnp.zeros((_NPAD - _N, _F_IN), jnp.float32)], axis=0)

    deg_parts = _deg_kernel(dst_flat, zeros_n)
    dinvb, hs, hsb = _tc_first(deg_parts, feat_pad, W1)

    acc = _spmm_kernel(src_p, dst_p, hsb, zrows)
    hs, hsb = _tc_mid(acc, hs, dinvb, W2, b1.reshape(1, _H))

    acc = _spmm_kernel(src_p, dst_p, hsb, zrows)
    hs, hsb = _tc_mid(acc, hs, dinvb, W3, b2.reshape(1, _H))

    acc = _spmm_kernel(src_p, dst_p, hsb, zrows)
    out = _tc_fin(acc, hs, dinvb, b3.reshape(1, _H))
    return out[:_N]


# speedup vs baseline: 1.1412x; 1.1341x over previous
import functools

import jax
import jax.numpy as jnp
from jax import lax
from jax.experimental import pallas as pl
from jax.experimental.pallas import tpu as pltpu
from jax.experimental.pallas import tpu_sc as plsc

_N, _E, _F_IN, _H = 10000, 320000, 128, 64

_NC, _NS = 2, 16
_NW = _NC * _NS
_CH = 128
_EPW = 10240
_NCHUNK = _EPW // _CH
_EPAD = _NW * _EPW
_NPAD = 10240
_NG = 10
_RPT = _NPAD // _NS
_BLK = 640

_mesh = plsc.VectorSubcoreMesh(core_axis_name="c", subcore_axis_name="s")


@functools.partial(
    pl.kernel,
    out_type=jax.ShapeDtypeStruct((_NW, _NPAD), jnp.float32),
    mesh=_mesh,
    compiler_params=pltpu.CompilerParams(needs_layout_passes=False),
    scratch_types=[
        pltpu.VMEM((_EPW,), jnp.int32),
        pltpu.VMEM((_NPAD,), jnp.float32),
    ],
)
def _deg_kernel(dst_hbm, zeros_hbm, out_hbm, dst_v, deg_v):
    c = lax.axis_index("c")
    s = lax.axis_index("s")
    w = s * _NC + c
    pltpu.sync_copy(dst_hbm.at[w], dst_v)
    pltpu.sync_copy(zeros_hbm, deg_v)
    ones = jnp.ones((16,), jnp.float32)

    def body(i, carry):
        idx = dst_v[pl.ds(i * 16, 16)]
        plsc.addupdate_scatter(deg_v, [idx], ones)
        return carry

    lax.fori_loop(0, _EPW // 16, body, 0)
    pltpu.sync_copy(deg_v, out_hbm.at[w])


@functools.partial(
    pl.kernel,
    out_type=jax.ShapeDtypeStruct((_NC, _NPAD, _H), jnp.float32),
    mesh=_mesh,
    compiler_params=pltpu.CompilerParams(
        needs_layout_passes=False, use_tc_tiling_on_sc=False),
    scratch_types=[
        pltpu.VMEM((_NCHUNK, _CH), jnp.int32),
        pltpu.VMEM((_NCHUNK, _CH), jnp.int32),
        pltpu.VMEM((_NG, _CH, _H // 2), jnp.int32),
        pltpu.VMEM((2, _CH, _H), jnp.float32),
        pltpu.VMEM_SHARED((_NPAD, _H), jnp.float32),
    ] + [pltpu.SemaphoreType.DMA] * (_NG + 2),
)
def _spmm_kernel(src_hbm, dst_hbm, hsb_hbm, zrows_hbm, out_hbm,
                 src_v, dst_v, rows_b, rows_f, acc_sh, *sems):
    c = lax.axis_index("c")
    s = lax.axis_index("s")
    w = s * _NC + c
    pltpu.sync_copy(src_hbm.at[w], src_v)
    pltpu.sync_copy(dst_hbm.at[w], dst_v)
    pltpu.sync_copy(zrows_hbm, acc_sh.at[pl.ds(s * _RPT, _RPT)])
    plsc.subcore_barrier()

    gsems = sems[:_NG]
    ssems = sems[_NG:]

    def fire_g(buf, g):
        pltpu.async_copy(hsb_hbm.at[src_v.at[g]], rows_b.at[buf], gsems[buf])

    def drain_g(buf, g):
        pltpu.make_async_copy(hsb_hbm.at[src_v.at[g]], rows_b.at[buf],
                              gsems[buf]).wait()

    def unpack(buf, sbuf):
        rb = rows_b.at[buf]
        rf = rows_f.at[sbuf]

        @pl.loop(0, _CH, unroll=8)
        def _rows(r):
            for g in range(_H // 32):
                x = rb[r, pl.ds(g * 16, 16)]
                a = plsc.bitcast(x << 16, jnp.float32)
                b = plsc.bitcast(x & jnp.int32(-65536), jnp.float32)
                rf[r, pl.ds(g * 16, 16)] = a
                rf[r, pl.ds(_H // 2 + g * 16, 16)] = b

    def fire_s(sbuf, g):
        pltpu.async_copy(rows_f.at[sbuf], acc_sh.at[dst_v.at[g]], ssems[sbuf],
                         add=True)

    def drain_s(sbuf, g):
        pltpu.make_async_copy(rows_f.at[sbuf], acc_sh.at[dst_v.at[g]],
                              ssems[sbuf]).wait()

    for i in range(_NG):
        fire_g(i, i)

    @pl.loop(0, _NCHUNK, step=_NG)
    def _pipeline(t):
        for i in range(_NG):
            g = t + i
            drain_g(i, g)

            @pl.when(g >= 2)
            def _():
                drain_s(i % 2, g - 2)

            unpack(i, i % 2)
            fire_s(i % 2, g)

            @pl.when(g + _NG < _NCHUNK)
            def _():
                fire_g(i, g + _NG)

    drain_s(0, _NCHUNK - 2)
    drain_s(1, _NCHUNK - 1)
    plsc.subcore_barrier()
    pltpu.sync_copy(acc_sh.at[pl.ds(s * _RPT, _RPT)],
                    out_hbm.at[c, pl.ds(s * _RPT, _RPT)])


def _shuffle_bf16(x):
    ua = lax.bitcast_convert_type(x[:, : _H // 2], jnp.int32)
    ub = lax.bitcast_convert_type(x[:, _H // 2 :], jnp.int32)

    def rne16(u):
        return ((u + 0x7FFF + ((u >> 16) & 1)) >> 16) & 0xFFFF

    packed = rne16(ua) | (rne16(ub) << 16)
    return packed


def _first_body(deg_ref, x_ref, w_ref, dinv_ref, hs_ref, hsb_ref):
    deg = jnp.sum(deg_ref[...], axis=0) + 1.0
    dinv = lax.rsqrt(deg)
    dinvb = jnp.broadcast_to(dinv[:, None], (_BLK, _H))
    dinv_ref[...] = dinvb
    h = jnp.dot(x_ref[...], w_ref[...], preferred_element_type=jnp.float32)
    hs = h * dinvb
    hs_ref[...] = hs
    hsb_ref[...] = _shuffle_bf16(hs)


def _mid_body(acc_ref, hs_ref, dinv_ref, w_ref, b_ref, out_ref, hsb_ref):
    acc = acc_ref[...]
    z = (acc[0] + acc[1] + hs_ref[...]) * dinv_ref[...] + b_ref[...]
    z = jnp.maximum(z, 0.0)
    hsn = (jnp.dot(z, w_ref[...], preferred_element_type=jnp.float32)
           * dinv_ref[...])
    out_ref[...] = hsn
    hsb_ref[...] = _shuffle_bf16(hsn)


def _fin_body(acc_ref, hs_ref, dinv_ref, b_ref, out_ref):
    acc = acc_ref[...]
    z = (acc[0] + acc[1] + hs_ref[...]) * dinv_ref[...] + b_ref[...]
    m = jnp.max(z, axis=1, keepdims=True)
    e = jnp.exp(z - m)
    lse = jnp.log(jnp.sum(e, axis=1, keepdims=True))
    out_ref[...] = z - m - lse


def _tc_first(deg_parts, feat, w1):
    return pl.pallas_call(
        _first_body,
        grid=(_NPAD // _BLK,),
        in_specs=[
            pl.BlockSpec((_NW, _BLK), lambda i: (0, i)),
            pl.BlockSpec((_BLK, _F_IN), lambda i: (i, 0)),
            pl.BlockSpec((_F_IN, _H), lambda i: (0, 0)),
        ],
        out_specs=[
            pl.BlockSpec((_BLK, _H), lambda i: (i, 0)),
            pl.BlockSpec((_BLK, _H), lambda i: (i, 0)),
            pl.BlockSpec((_BLK, _H // 2), lambda i: (i, 0)),
        ],
        out_shape=[
            jax.ShapeDtypeStruct((_NPAD, _H), jnp.float32),
            jax.ShapeDtypeStruct((_NPAD, _H), jnp.float32),
            jax.ShapeDtypeStruct((_NPAD, _H // 2), jnp.int32),
        ],
    )(deg_parts, feat, w1)


def _tc_mid(acc, hs, dinvb, w, b):
    return pl.pallas_call(
        _mid_body,
        grid=(_NPAD // _BLK,),
        in_specs=[
            pl.BlockSpec((_NC, _BLK, _H), lambda i: (0, i, 0)),
            pl.BlockSpec((_BLK, _H), lambda i: (i, 0)),
            pl.BlockSpec((_BLK, _H), lambda i: (i, 0)),
            pl.BlockSpec((_H, _H), lambda i: (0, 0)),
            pl.BlockSpec((1, _H), lambda i: (0, 0)),
        ],
        out_specs=[
            pl.BlockSpec((_BLK, _H), lambda i: (i, 0)),
            pl.BlockSpec((_BLK, _H // 2), lambda i: (i, 0)),
        ],
        out_shape=[
            jax.ShapeDtypeStruct((_NPAD, _H), jnp.float32),
            jax.ShapeDtypeStruct((_NPAD, _H // 2), jnp.int32),
        ],
    )(acc, hs, dinvb, w, b)


def _tc_fin(acc, hs, dinvb, b):
    return pl.pallas_call(
        _fin_body,
        grid=(_NPAD // _BLK,),
        in_specs=[
            pl.BlockSpec((_NC, _BLK, _H), lambda i: (0, i, 0)),
            pl.BlockSpec((_BLK, _H), lambda i: (i, 0)),
            pl.BlockSpec((_BLK, _H), lambda i: (i, 0)),
            pl.BlockSpec((1, _H), lambda i: (0, 0)),
        ],
        out_specs=pl.BlockSpec((_BLK, _H), lambda i: (i, 0)),
        out_shape=jax.ShapeDtypeStruct((_NPAD, _H), jnp.float32),
    )(acc, hs, dinvb, b)


def kernel(edges, features, W1, b1, W2, b2, W3, b3):
    src = edges[0].astype(jnp.int32)
    dst = edges[1].astype(jnp.int32)
    src_p = jnp.concatenate(
        [src, jnp.zeros((_EPAD - _E,), jnp.int32)]).reshape(_NW, _NCHUNK, _CH)
    dst_p = jnp.concatenate(
        [dst, jnp.full((_EPAD - _E,), _N, jnp.int32)]).reshape(_NW, _NCHUNK, _CH)
    dst_flat = dst_p.reshape(_NW, _EPW)
    zeros_n = jnp.zeros((_NPAD,), jnp.float32)
    zrows = jnp.zeros((_RPT, _H), jnp.float32)
    feat_pad = jnp.concatenate(
        [features, jnp.zeros((_NPAD - _N, _F_IN), jnp.float32)], axis=0)

    deg_parts = _deg_kernel(dst_flat, zeros_n)
    dinvb, hs, hsb = _tc_first(deg_parts, feat_pad, W1)

    acc = _spmm_kernel(src_p, dst_p, hsb, zrows)
    hs, hsb = _tc_mid(acc, hs, dinvb, W2, b1.reshape(1, _H))

    acc = _spmm_kernel(src_p, dst_p, hsb, zrows)
    hs, hsb = _tc_mid(acc, hs, dinvb, W3, b2.reshape(1, _H))

    acc = _spmm_kernel(src_p, dst_p, hsb, zrows)
    out = _tc_fin(acc, hs, dinvb, b3.reshape(1, _H))
    return out[:_N]
